# Initial kernel scaffold; baseline (speedup 1.0000x reference)
import jax
import jax.numpy as jnp
from jax.experimental import pallas as pl


def kernel(rpn_scores, rpn_reg, xyz):
    raise NotImplementedError("write your pallas kernel here")


# speedup vs baseline: 22.0819x; 22.0819x over previous
import jax
import jax.numpy as jnp
import numpy as np
from jax.experimental import pallas as pl

_LOC_SCOPE = 3.0
_LOC_BIN_SIZE = 0.5
_NUM_HEAD_BIN = 12
_NMS_POST = 512
_NMS_THRES = 0.85
_MEAN_SIZE = (1.52563191, 1.62856739, 3.8831164)
_NB = 12


def _argmax_rows(ref, base):
    mv = ref[base]
    mi = jnp.zeros_like(mv)
    for b in range(1, _NB):
        v = ref[base + b]
        gt = v > mv
        mv = jnp.where(gt, v, mv)
        mi = jnp.where(gt, jnp.float32(b), mi)
    return mi


def _gather_rows(ref, base, mi):
    acc = jnp.zeros_like(mi)
    for b in range(_NB):
        acc = acc + jnp.where(mi == jnp.float32(b), ref[base + b], 0.0)
    return acc


def _decode_kernel(reg_ref, xyz_ref, feat_ref):
    xi = _argmax_rows(reg_ref, 0)
    zi = _argmax_rows(reg_ref, _NB)
    xr = _gather_rows(reg_ref, 2 * _NB, xi)
    zr = _gather_rows(reg_ref, 3 * _NB, zi)
    half = _LOC_BIN_SIZE / 2.0 - _LOC_SCOPE
    px = xi * _LOC_BIN_SIZE + half + xr * _LOC_BIN_SIZE + xyz_ref[0]
    pz = zi * _LOC_BIN_SIZE + half + zr * _LOC_BIN_SIZE + xyz_ref[2]
    py = xyz_ref[1] + reg_ref[48]
    ri = _argmax_rows(reg_ref, 49)
    rr = _gather_rows(reg_ref, 61, ri)
    apc = 2.0 * np.pi / _NUM_HEAD_BIN
    ry = ri * apc + rr * (apc / 2.0)
    two_pi = jnp.float32(2.0 * np.pi)
    ry = ry - jnp.floor(ry / two_pi) * two_pi
    ry = jnp.where(ry > np.pi, ry - two_pi, ry)
    h = reg_ref[73] * _MEAN_SIZE[0] + _MEAN_SIZE[0]
    w = reg_ref[74] * _MEAN_SIZE[1] + _MEAN_SIZE[1]
    l = reg_ref[75] * _MEAN_SIZE[2] + _MEAN_SIZE[2]
    x1 = px - h / 2.0
    y1 = pz - l / 2.0
    x2 = px + h / 2.0
    y2 = pz + l / 2.0
    area = jnp.maximum(x2 - x1, 0.0) * jnp.maximum(y2 - y1, 0.0)
    feat_ref[0] = px
    feat_ref[1] = py
    feat_ref[2] = pz
    feat_ref[3] = h
    feat_ref[4] = w
    feat_ref[5] = l
    feat_ref[6] = ry
    feat_ref[7] = x1
    feat_ref[8] = y1
    feat_ref[9] = x2
    feat_ref[10] = y2
    feat_ref[11] = area


def _nms_kernel(feat_ref, sc_ref, bbox_ref, sout_ref):
    nbatch, n = sc_ref.shape
    scores = sc_ref[...]
    x1 = feat_ref[7]
    y1 = feat_ref[8]
    x2 = feat_ref[9]
    y2 = feat_ref[10]
    area = feat_ref[11]
    props = [feat_ref[c] for c in range(7)]
    idx = jax.lax.broadcasted_iota(jnp.int32, (nbatch, n), 1)
    neg_inf = jnp.float32(-jnp.inf)

    one = jnp.float32(1.0)
    zero = jnp.float32(0.0)

    def body(t, supf):
        s = jnp.where(supf > 0.5, neg_inf, scores)
        m = jnp.max(s, axis=1, keepdims=True)
        validf = jnp.where(m > neg_inf, one, zero)
        j = jnp.min(jnp.where(s == m, idx, jnp.int32(n)), axis=1,
                    keepdims=True)
        ohf = jnp.where(idx == j, one, zero)

        def sel(a):
            return jnp.sum(ohf * a, axis=1, keepdims=True)

        sx1 = sel(x1)
        sy1 = sel(y1)
        sx2 = sel(x2)
        sy2 = sel(y2)
        sar = sel(area)
        xx1 = jnp.maximum(sx1, x1)
        yy1 = jnp.maximum(sy1, y1)
        xx2 = jnp.minimum(sx2, x2)
        yy2 = jnp.minimum(sy2, y2)
        inter = jnp.maximum(xx2 - xx1, 0.0) * jnp.maximum(yy2 - yy1, 0.0)
        iou = inter / jnp.maximum(sar + area - inter, 1e-8)
        iouf = jnp.where(iou > _NMS_THRES, one, zero)
        sup_new = jnp.maximum(jnp.maximum(supf, iouf), ohf)
        supf = supf + validf * (sup_new - supf)
        for c in range(7):
            v = sel(props[c]) * validf
            bbox_ref[:, pl.ds(t, 1), pl.ds(c, 1)] = v.reshape(nbatch, 1, 1)
        sv = sel(scores) * validf
        sout_ref[:, pl.ds(t, 1), pl.ds(0, 1)] = sv.reshape(nbatch, 1, 1)
        return supf

    jax.lax.fori_loop(0, _NMS_POST, body,
                      jnp.zeros((nbatch, n), dtype=jnp.float32))


def kernel(rpn_scores, rpn_reg, xyz):
    batch, n, _ = rpn_scores.shape
    total = batch * n
    rows = total // 8
    reg_t = rpn_reg.reshape(total, 76).T.reshape(76, 8, rows)
    xyz_t = xyz.reshape(total, 3).T.reshape(3, 8, rows)
    chunk = 2048
    nchunks = rows // chunk
    feat = pl.pallas_call(
        _decode_kernel,
        grid=(nchunks,),
        in_specs=[
            pl.BlockSpec((76, 8, chunk), lambda i: (0, 0, i)),
            pl.BlockSpec((3, 8, chunk), lambda i: (0, 0, i)),
        ],
        out_specs=pl.BlockSpec((12, 8, chunk), lambda i: (0, 0, i)),
        out_shape=jax.ShapeDtypeStruct((12, 8, rows), jnp.float32),
    )(reg_t, xyz_t)
    feat4 = feat.reshape(12, batch, n)
    sc = rpn_scores.reshape(batch, n)
    bbox8, sout8 = pl.pallas_call(
        _nms_kernel,
        out_shape=[
            jax.ShapeDtypeStruct((batch, _NMS_POST, 8), jnp.float32),
            jax.ShapeDtypeStruct((batch, _NMS_POST, 8), jnp.float32),
        ],
    )(feat4, sc)
    return bbox8[:, :, :7], sout8[:, :, :1]
